# baseline (speedup 1.0000x reference)
import functools
import math

import jax
import jax.numpy as jnp
from jax import lax
from jax.experimental import pallas as pl
from jax.experimental.pallas import tpu as pltpu
from jax.experimental.pallas import tpu_sc as plsc

_SEQ = 4096
_BATCH = 4
_DIM = 64
_VOCAB = 1000000
_SCALE = math.sqrt(_DIM)

_NW = 32
_STRIPE = 31232
_CHW = 256
_NCH = _STRIPE // _CHW
_ROWS = _SEQ * _BATCH
_TRASH = _ROWS
_OUT1R = _ROWS + 8

_MESH = plsc.VectorSubcoreMesh(core_axis_name="c", subcore_axis_name="s")
_PARAMS = pltpu.CompilerParams(use_tc_tiling_on_sc=True, needs_layout_passes=False)


def _iota16():
    return lax.iota(jnp.int32, 16)


def _count(m):
    return jnp.sum(jnp.where(m, 1, 0))


@functools.partial(
    pl.kernel,
    out_type=jax.ShapeDtypeStruct((_OUT1R, 128), jnp.float32),
    mesh=_MESH,
    scratch_types=[
        pltpu.VMEM((_BATCH, _SEQ), jnp.int32),
        pltpu.VMEM((_ROWS,), jnp.int32),
        pltpu.VMEM((_ROWS,), jnp.int32),
        pltpu.VMEM((_DIM, _CHW), jnp.float32),
        pltpu.VMEM((_DIM, _CHW), jnp.float32),
        pltpu.VMEM((_DIM, 64), jnp.float32),
        pltpu.VMEM((128, 16), jnp.float32),
        pltpu.VMEM((16, 128), jnp.float32),
        pltpu.VMEM((16,), jnp.int32),
        pltpu.SemaphoreType.DMA,
        pltpu.SemaphoreType.DMA,
        pltpu.SemaphoreType.DMA,
    ],
    compiler_params=_PARAMS,
)
def _gather_pass(x_hbm, tab_hbm, tail_hbm, out_hbm, idxv, fbuf, cbuf, chv0,
                 chv1, tailv, abuf, rowb, flist, sem0, sem1, sem2):
    wid = lax.axis_index("s") * 2 + lax.axis_index("c")
    wlo = wid * _STRIPE
    whi = jnp.where(wid == _NW - 1, _VOCAB, wlo + _STRIPE)
    i16 = _iota16()

    pltpu.sync_copy(x_hbm, idxv)

    def prefilter_b(b, n):
        def scan_g(g, n):
            ms, fs, cnts = [], [], []
            for k in range(8):
                off = g * 128 + k * 16
                v = idxv[b, pl.ds(off, 16)]
                f = (off + i16) * _BATCH + b
                m = (v >= wlo) & (v < whi)
                ms.append(m)
                fs.append(f)
                cnts.append(_count(m))
            for k in range(8):
                plsc.store_compressed(fbuf.at[pl.ds(n, 16)], fs[k], mask=ms[k])
                n = n + cnts[k]
            return n
        return lax.fori_loop(0, _SEQ // 128, scan_g, n)

    n = lax.fori_loop(0, _BATCH, prefilter_b, 0)

    def scan_extract(src_ref, cs, hi):
        def scan_q(q, cnt):
            ms, fs, cnts = [], [], []
            for k in range(8):
                off = q * 128 + k * 16
                fr = fbuf[pl.ds(off, 16)]
                valid = (off + i16) < n
                fq = jnp.where(valid, fr, 0)
                v = plsc.load_gather(idxv, [fq & 3, fq >> 2], mask=valid)
                m = valid & (v >= cs) & (v < hi)
                ms.append(m)
                fs.append(fq)
                cnts.append(_count(m))
            for k in range(8):
                plsc.store_compressed(cbuf.at[pl.ds(cnt, 16)], fs[k], mask=ms[k])
                cnt = cnt + cnts[k]
            return cnt

        cnt = lax.fori_loop(0, (n + 127) // 128, scan_q, 0)

        def ext(e, carry):
            fq_r = cbuf[pl.ds(e * 16, 16)]
            valid = (e * 16 + i16) < cnt
            fq = jnp.where(valid, fq_r, 0)
            v = plsc.load_gather(idxv, [fq & 3, fq >> 2], mask=valid)
            lv = jnp.where(valid, v - cs, 0)
            for d in range(_DIM):
                g = plsc.load_gather(src_ref, [jnp.full((16,), d, jnp.int32), lv],
                                     mask=valid)
                abuf[d, :] = g
            flist[...] = jnp.where(valid, fq_r, _TRASH)
            for j in range(16):
                for cg in range(_DIM // 16):
                    rowb[j, pl.ds(cg * 16, 16)] = plsc.load_gather(
                        abuf, [cg * 16 + i16, jnp.full((16,), j, jnp.int32)])
            pltpu.async_copy(rowb, out_hbm.at[flist], sem2).wait()
            return carry

        lax.fori_loop(0, (cnt + 15) // 16, ext, 0)

    nch = _NCH + jnp.where(wid == _NW - 1, 1, 0)

    pltpu.async_copy(tab_hbm.at[:, pl.ds(wlo, _CHW)], chv0, sem0)
    pltpu.async_copy(tab_hbm.at[:, pl.ds(wlo + _CHW, _CHW)], chv1, sem1)

    def outer(i, carry):
        for b, (chvb, semb) in enumerate(((chv0, sem0), (chv1, sem1))):
            c = i * 2 + b

            @pl.when(c < nch)
            def _():
                cs = pl.multiple_of(wlo + c * _CHW, 128)
                pltpu.make_async_copy(
                    tab_hbm.at[:, pl.ds(cs, _CHW)], chvb, semb).wait()
                scan_extract(chvb, cs, cs + _CHW)

                @pl.when(c + 2 < nch)
                def _():
                    cs2 = pl.multiple_of(wlo + (c + 2) * _CHW, 128)
                    pltpu.async_copy(
                        tab_hbm.at[:, pl.ds(cs2, _CHW)], chvb, semb)
        return carry

    lax.fori_loop(0, (_NCH + 2) // 2, outer, 0)

    @pl.when(wid == _NW - 1)
    def _():
        pltpu.sync_copy(tail_hbm, tailv)
        scan_extract(tailv, _VOCAB - 64, _VOCAB)


@functools.partial(
    pl.kernel,
    out_type=jax.ShapeDtypeStruct((_BATCH, _DIM, _SEQ), jnp.float32),
    mesh=_MESH,
    scratch_types=[
        pltpu.VMEM((512, 128), jnp.float32),
        pltpu.VMEM((_DIM, 128), jnp.float32),
        pltpu.VMEM((_BATCH, _DIM, 128), jnp.float32),
    ],
    compiler_params=_PARAMS,
)
def _finish_pass(rows_hbm, pe_hbm, out_hbm, rv, pv, ov):
    wid = lax.axis_index("s") * 2 + lax.axis_index("c")
    s0 = wid * 128
    i16 = _iota16()

    pltpu.sync_copy(rows_hbm.at[pl.ds(wid * 512, 512)], rv)
    pltpu.sync_copy(pe_hbm.at[:, pl.ds(s0, 128)], pv)

    def body(d, carry):
        dsplat = jnp.full((16,), d, jnp.int32)
        for b in range(_BATCH):
            for sg in range(128 // 16):
                fl = (sg * 16 + i16) * _BATCH + b
                raw = plsc.load_gather(rv, [fl, dsplat])
                ov[b, d, pl.ds(sg * 16, 16)] = raw * _SCALE + pv[d, pl.ds(sg * 16, 16)]
        return carry

    lax.fori_loop(0, _DIM, body, 0)

    pltpu.sync_copy(ov, out_hbm.at[:, :, pl.ds(s0, 128)])


def kernel(x, table, pe):
    tab_t = table.T
    raw = _gather_pass(x.T, tab_t, tab_t[:, _VOCAB - 64:])
    out_t = _finish_pass(raw, pe[:, 0, :].T)
    return jnp.transpose(out_t, (2, 0, 1))


# speedup vs baseline: 3.2983x; 3.2983x over previous
import functools
import math

import jax
import jax.numpy as jnp
from jax import lax
from jax.experimental import pallas as pl
from jax.experimental.pallas import tpu as pltpu
from jax.experimental.pallas import tpu_sc as plsc

_SEQ = 4096
_BATCH = 4
_DIM = 64
_SCALE = math.sqrt(_DIM)
_SPW = 512
_LANES = 16


@functools.partial(
    pl.kernel,
    out_type=jax.ShapeDtypeStruct((_BATCH, _DIM, _SEQ), jnp.float32),
    mesh=plsc.VectorSubcoreMesh(core_axis_name="c", subcore_axis_name="s"),
    scratch_types=[
        pltpu.VMEM((_SPW,), jnp.int32),
        pltpu.VMEM((_SPW, _DIM), jnp.float32),
        pltpu.VMEM((_DIM, _SPW), jnp.float32),
        pltpu.VMEM((_DIM, _SPW), jnp.float32),
        pltpu.SemaphoreType.DMA,
    ],
    compiler_params=pltpu.CompilerParams(use_tc_tiling_on_sc=False,
                                         needs_layout_passes=False),
)
def _pe_embed2(x_hbm, tab_hbm, pe_hbm, out_hbm, idx_v, rows_v, pe_v, outb, sem):
    wid = lax.axis_index("s") * 2 + lax.axis_index("c")
    b = wid % _BATCH
    s0 = (wid // _BATCH) * _SPW
    i16 = lax.iota(jnp.int32, _LANES)

    pltpu.sync_copy(x_hbm.at[b, pl.ds(s0, _SPW)], idx_v)

    copies = []
    for j in range(_SPW // 128):
        copies.append(
            pltpu.async_copy(
                tab_hbm.at[idx_v.at[pl.ds(j * 128, 128)]],
                rows_v.at[pl.ds(j * 128, 128)],
                sem,
            )
        )
    pltpu.sync_copy(pe_hbm.at[:, pl.ds(s0, _SPW)], pe_v)
    for cp in copies:
        cp.wait()

    def body(d, carry):
        dsplat = jnp.full((_LANES,), d, jnp.int32)
        for g in range(_SPW // _LANES):
            sl = pl.ds(g * _LANES, _LANES)
            raw = plsc.load_gather(rows_v, [g * _LANES + i16, dsplat])
            outb[d, sl] = raw * _SCALE + pe_v[d, sl]
        return carry

    lax.fori_loop(0, _DIM, body, 0)

    pltpu.sync_copy(outb, out_hbm.at[b, :, pl.ds(s0, _SPW)])


def kernel(x, table, pe):
    out_t = _pe_embed2(x.T, table, pe[:, 0, :].T)
    return jnp.transpose(out_t, (2, 0, 1))


# speedup vs baseline: 3.3949x; 1.0293x over previous
import functools
import math

import jax
import jax.numpy as jnp
from jax import lax
from jax.experimental import pallas as pl
from jax.experimental.pallas import tpu as pltpu
from jax.experimental.pallas import tpu_sc as plsc

_SEQ = 4096
_BATCH = 4
_DIM = 64
_SCALE = math.sqrt(_DIM)

_NC = 2
_NS = 16
_NW = _NC * _NS
_ROWS = _SEQ * _BATCH
_RPW = _ROWS // _NW
_SPW = _SEQ // _NW
_CHUNK = 128
_NCHUNK = _RPW // _CHUNK
_LANES = 16


@functools.partial(
    pl.kernel,
    out_type=jax.ShapeDtypeStruct((_ROWS, _DIM), jnp.float32),
    mesh=plsc.VectorSubcoreMesh(core_axis_name="c", subcore_axis_name="s"),
    scratch_types=[
        pltpu.VMEM((_NCHUNK, _CHUNK), jnp.int32),
        pltpu.VMEM((_RPW, _DIM), jnp.float32),
        pltpu.VMEM((_SPW, _DIM), jnp.float32),
        pltpu.SemaphoreType.DMA,
    ],
    compiler_params=pltpu.CompilerParams(use_tc_tiling_on_sc=False),
)
def _pe_embed(idx_hbm, table_hbm, pe_hbm, out_hbm, idx_v, rows_v, pe_v, sem):
    wid = lax.axis_index("s") * _NC + lax.axis_index("c")
    base = wid * _RPW
    sbase = wid * _SPW

    pltpu.sync_copy(idx_hbm.at[pl.ds(wid * _NCHUNK, _NCHUNK)], idx_v)

    copies = []
    for j in range(_NCHUNK):
        copies.append(
            pltpu.async_copy(
                table_hbm.at[idx_v.at[j]],
                rows_v.at[pl.ds(j * _CHUNK, _CHUNK)],
                sem,
            )
        )
    pltpu.sync_copy(pe_hbm.at[pl.ds(sbase, _SPW)], pe_v)
    for cp in copies:
        cp.wait()

    def body(s, carry):
        r0 = s * _BATCH
        for c in range(_DIM // _LANES):
            pvec = pe_v[s, pl.ds(c * _LANES, _LANES)]
            for b in range(_BATCH):
                rv = rows_v[r0 + b, pl.ds(c * _LANES, _LANES)]
                rows_v[r0 + b, pl.ds(c * _LANES, _LANES)] = rv * _SCALE + pvec
        return carry

    lax.fori_loop(0, _SPW, body, 0)

    pltpu.sync_copy(rows_v, out_hbm.at[pl.ds(base, _RPW)])


def kernel(x, table, pe):
    idx2d = x.reshape(_NW * _NCHUNK, _CHUNK)
    pe2d = pe[:_SEQ, 0, :]
    out = _pe_embed(idx2d, table, pe2d)
    return out.reshape(_SEQ, _BATCH, _DIM)
